# Initial kernel scaffold; baseline (speedup 1.0000x reference)
import jax
import jax.numpy as jnp
from jax.experimental import pallas as pl


def kernel(x, labels, times):
    raise NotImplementedError("write your pallas kernel here")


# speedup vs baseline: 5.0193x; 5.0193x over previous
import functools

import jax
import jax.numpy as jnp
from jax import lax
from jax.experimental import pallas as pl
from jax.experimental.pallas import tpu as pltpu
from jax.experimental.pallas import tpu_sc as plsc

N = 100000
D = 128
NUM_TIME = 20
NUM_LABEL = 16
SPLIT = 15
NSEG = NUM_TIME * NUM_LABEL
STRIDE = D + 16
ACC = NSEG * STRIDE

NC, NS, L = 2, 16, 16
NW = NC * NS
C = 160
NCHUNK = N // C
CW = C * D
MAXCH = (NCHUNK + NW - 1) // NW

_mesh = plsc.VectorSubcoreMesh(
    core_axis_name="c", subcore_axis_name="s", num_cores=NC, num_subcores=NS)
_sc_params = pltpu.CompilerParams(needs_layout_passes=False)


def _lane():
    return lax.iota(jnp.int32, L)


def _splat(buf, r):
    return plsc.load_gather(buf, [jnp.full((L,), r, jnp.int32)])


@functools.partial(
    pl.kernel,
    out_type=jax.ShapeDtypeStruct((NW, ACC), jnp.float32),
    mesh=_mesh,
    scratch_types=[
        pltpu.VMEM((CW,), jnp.float32),
        pltpu.VMEM((C,), jnp.int32),
        pltpu.VMEM((C,), jnp.int32),
        pltpu.VMEM((ACC,), jnp.float32),
        pltpu.VMEM((L,), jnp.int32),
        pltpu.VMEM((L,), jnp.float32),
    ],
    compiler_params=_sc_params,
)
def _pass1(x_hbm, labels_hbm, times_hbm, out_hbm, xb, lb, tb, acc, sbuf, fbuf):
    wid = lax.axis_index("s") * NC + lax.axis_index("c")
    lane = _lane()

    def zero_body(i, _):
        acc[pl.ds(i * L, L)] = jnp.zeros((L,), jnp.float32)
        return 0
    lax.fori_loop(0, ACC // L, zero_body, 0)

    def chunk_body(i, _):
        j = wid + i * NW

        @pl.when(j < NCHUNK)
        def _():
            pltpu.sync_copy(x_hbm.at[pl.ds(j * CW, CW)], xb)
            pltpu.sync_copy(labels_hbm.at[pl.ds(j * C, C)], lb)
            pltpu.sync_copy(times_hbm.at[pl.ds(j * C, C)], tb)

            def group_body(g, _):
                lv = lb[pl.ds(g * L, L)]
                tv = tb[pl.ds(g * L, L)]
                seg = tv * NUM_LABEL + lv
                sbuf[...] = seg
                for r in range(L):
                    fb = _splat(sbuf, r) * STRIDE
                    base = g * (L * D) + r * D
                    sq = jnp.zeros((L,), jnp.float32)
                    for k in range(D // L):
                        xv = xb[pl.ds(base + k * L, L)]
                        plsc.addupdate_scatter(acc, [fb + (k * L) + lane], xv)
                        sq = sq + xv * xv
                    fbuf[...] = plsc.cumsum(sq)
                    rowsq = _splat(fbuf, L - 1)
                    vec2 = jnp.where(lane == 0, 1.0,
                                     jnp.where(lane == 1, rowsq, 0.0))
                    plsc.addupdate_scatter(acc, [fb + D + lane], vec2,
                                           mask=lane < 2)
                return 0
            lax.fori_loop(0, C // L, group_body, 0)
        return 0
    lax.fori_loop(0, MAXCH, chunk_body, 0)

    pltpu.sync_copy(acc, out_hbm.at[wid])


def _stats_body(p_ref, tbl_ref):
    f32 = jnp.float32
    ps = jnp.sum(p_ref[...], axis=0)
    sums = ps[:, :D]
    cnt = ps[:, D:D + 1]
    sumsq = ps[:, D + 1:D + 2]

    seg_t = lax.broadcasted_iota(jnp.int32, (NSEG, 1), 0) // NUM_LABEL
    G = (lax.broadcasted_iota(jnp.int32, (NUM_TIME, NSEG), 1) // NUM_LABEL
         == lax.broadcasted_iota(jnp.int32, (NUM_TIME, NSEG), 0)).astype(f32)
    GT = (lax.broadcasted_iota(jnp.int32, (NSEG, NUM_TIME), 0) // NUM_LABEL
          == lax.broadcasted_iota(jnp.int32, (NSEG, NUM_TIME), 1)).astype(f32)

    time_cnt = jnp.dot(G, cnt, preferred_element_type=f32)
    tsums = jnp.dot(G, sums, preferred_element_type=f32)
    mean = sums / jnp.maximum(1.0, cnt)
    ttm = tsums / jnp.maximum(1.0, time_cnt)
    ttm_seg = jnp.dot(GT, ttm, preferred_element_type=f32)

    diff = mean - ttm_seg
    msq_seg = cnt * jnp.sum(diff * diff, axis=1, keepdims=True)
    rsq_seg = (sumsq - 2.0 * jnp.sum(mean * sums, axis=1, keepdims=True)
               + cnt * jnp.sum(mean * mean, axis=1, keepdims=True))
    msq_t = jnp.dot(G, msq_seg, preferred_element_type=f32)
    rsq_t = jnp.dot(G, rsq_seg, preferred_element_type=f32)

    testm = (seg_t >= SPLIT).astype(f32)
    test_cnt = jnp.sum(testm * cnt)
    test_sum = jnp.sum(sums * testm, axis=0, keepdims=True)
    test_sumsq = jnp.sum(sumsq * testm)
    test_mean = test_sum / jnp.maximum(1.0, test_cnt)
    test_var = (test_sumsq - 2.0 * jnp.sum(test_mean * test_sum)
                + test_cnt * jnp.sum(test_mean * test_mean)
                ) / jnp.maximum(1.0, test_cnt - 1.0)

    t_iota = lax.broadcasted_iota(jnp.int32, (NUM_TIME, 1), 0)
    tmask = t_iota < SPLIT
    denom = jnp.maximum(1.0, time_cnt - 1.0)
    msq = jnp.where(tmask, msq_t / denom, msq_t)
    rsq = jnp.where(tmask, rsq_t / denom, rsq_t)
    alpha_sq = (test_var - msq) / jnp.maximum(1e-06, rsq)
    alpha = jnp.where(alpha_sq > 0,
                      jnp.sqrt(jnp.where(alpha_sq > 0, alpha_sq, 1.0)), 0.0)

    alpha_seg = jnp.dot(GT, alpha, preferred_element_type=f32)
    train_seg = seg_t < SPLIT
    A = jnp.where(train_seg, alpha_seg, 1.0)
    B = jnp.where(train_seg, (1.0 - alpha_seg) * mean, 0.0)
    tbl_ref[...] = jnp.concatenate(
        [B, A, jnp.zeros((NSEG, STRIDE - D - 1), f32)], axis=1)


_stats = pl.pallas_call(
    _stats_body,
    out_shape=jax.ShapeDtypeStruct((NSEG, STRIDE), jnp.float32),
)


@functools.partial(
    pl.kernel,
    out_type=jax.ShapeDtypeStruct((N * D,), jnp.float32),
    mesh=_mesh,
    scratch_types=[
        pltpu.VMEM((CW,), jnp.float32),
        pltpu.VMEM((C,), jnp.int32),
        pltpu.VMEM((C,), jnp.int32),
        pltpu.VMEM((ACC,), jnp.float32),
        pltpu.VMEM((CW,), jnp.float32),
        pltpu.VMEM((L,), jnp.int32),
        pltpu.VMEM((L,), jnp.float32),
    ],
    compiler_params=_sc_params,
)
def _pass2(x_hbm, labels_hbm, times_hbm, tbl_hbm, out_hbm,
           xb, lb, tb, tblv, ob, sbuf, abuf):
    wid = lax.axis_index("s") * NC + lax.axis_index("c")
    lane = _lane()
    pltpu.sync_copy(tbl_hbm, tblv)

    def chunk_body(i, _):
        j = wid + i * NW

        @pl.when(j < NCHUNK)
        def _():
            pltpu.sync_copy(x_hbm.at[pl.ds(j * CW, CW)], xb)
            pltpu.sync_copy(labels_hbm.at[pl.ds(j * C, C)], lb)
            pltpu.sync_copy(times_hbm.at[pl.ds(j * C, C)], tb)

            def group_body(g, _):
                lv = lb[pl.ds(g * L, L)]
                tv = tb[pl.ds(g * L, L)]
                seg = tv * NUM_LABEL + lv
                sbuf[...] = seg
                abuf[...] = plsc.load_gather(tblv, [seg * STRIDE + D])
                for r in range(L):
                    a = _splat(abuf, r)
                    fb = _splat(sbuf, r) * STRIDE
                    base = g * (L * D) + r * D
                    for k in range(D // L):
                        xv = xb[pl.ds(base + k * L, L)]
                        bv = plsc.load_gather(tblv, [fb + (k * L) + lane])
                        ob[pl.ds(base + k * L, L)] = a * xv + bv
                return 0
            lax.fori_loop(0, C // L, group_body, 0)
            pltpu.sync_copy(ob, out_hbm.at[pl.ds(j * CW, CW)])
        return 0
    lax.fori_loop(0, MAXCH, chunk_body, 0)


def kernel(x, labels, times):
    x_flat = x.reshape(-1)
    partials = _pass1(x_flat, labels, times)
    tbl = _stats(partials.reshape(NW, NSEG, STRIDE))
    out_flat = _pass2(x_flat, labels, times, tbl.reshape(-1))
    return out_flat.reshape(N, D)


# speedup vs baseline: 5.3693x; 1.0697x over previous
import functools

import jax
import jax.numpy as jnp
from jax import lax
from jax.experimental import pallas as pl
from jax.experimental.pallas import tpu as pltpu
from jax.experimental.pallas import tpu_sc as plsc

N = 100000
D = 128
NUM_TIME = 20
NUM_LABEL = 16
SPLIT = 15
NSEG = NUM_TIME * NUM_LABEL
STRIDE = D + 32
ACC = NSEG * STRIDE

NC, NS, L = 2, 16, 16
NW = NC * NS
C = 160
NCHUNK = N // C
CW = C * D
MAXCH = (NCHUNK + NW - 1) // NW

_mesh = plsc.VectorSubcoreMesh(
    core_axis_name="c", subcore_axis_name="s", num_cores=NC, num_subcores=NS)
_sc_params = pltpu.CompilerParams(needs_layout_passes=False)


def _lane():
    return lax.iota(jnp.int32, L)


def _splat(buf, r):
    return plsc.load_gather(buf, [jnp.full((L,), r, jnp.int32)])


@functools.partial(
    pl.kernel,
    out_type=jax.ShapeDtypeStruct((NW, ACC), jnp.float32),
    mesh=_mesh,
    scratch_types=[
        pltpu.VMEM((CW,), jnp.float32),
        pltpu.VMEM((C,), jnp.int32),
        pltpu.VMEM((C,), jnp.int32),
        pltpu.VMEM((ACC,), jnp.float32),
        pltpu.VMEM((L,), jnp.int32),
        pltpu.VMEM((L,), jnp.float32),
    ],
    compiler_params=_sc_params,
)
def _pass1(x_hbm, labels_hbm, times_hbm, out_hbm, xb, lb, tb, acc, sbuf, fbuf):
    wid = lax.axis_index("s") * NC + lax.axis_index("c")
    lane = _lane()
    ones = jnp.ones((L,), jnp.float32)

    def zero_body(i, _):
        acc[pl.ds(i * L, L)] = jnp.zeros((L,), jnp.float32)
        return 0
    lax.fori_loop(0, ACC // L, zero_body, 0)

    def chunk_body(i, _):
        j = wid + i * NW

        @pl.when(j < NCHUNK)
        def _():
            pltpu.sync_copy(x_hbm.at[pl.ds(j * CW, CW)], xb)
            pltpu.sync_copy(labels_hbm.at[pl.ds(j * C, C)], lb)
            pltpu.sync_copy(times_hbm.at[pl.ds(j * C, C)], tb)

            def group_body(g, _):
                lv = lb[pl.ds(g * L, L)]
                tv = tb[pl.ds(g * L, L)]
                seg = tv * NUM_LABEL + lv
                sbuf[...] = seg
                for r in range(L):
                    fb = _splat(sbuf, r) * STRIDE
                    base = g * (L * D) + r * D
                    sq = jnp.zeros((L,), jnp.float32)
                    for k in range(D // L):
                        xv = xb[pl.ds(base + k * L, L)]
                        plsc.addupdate_scatter(acc, [fb + (k * L) + lane], xv)
                        sq = sq + xv * xv
                    plsc.addupdate_scatter(acc, [fb + D + lane], sq)
                    plsc.addupdate_scatter(acc, [fb + (D + L) + lane], ones)
                return 0
            lax.fori_loop(0, C // L, group_body, 0)
        return 0
    lax.fori_loop(0, MAXCH, chunk_body, 0)

    pltpu.sync_copy(acc, out_hbm.at[wid])


def _stats_body(p_ref, tbl_ref):
    f32 = jnp.float32
    ps = jnp.sum(p_ref[...], axis=0)
    sums = ps[:, :D]
    cnt = ps[:, D + L:D + L + 1]
    sumsq = jnp.sum(ps[:, D:D + L], axis=1, keepdims=True)

    seg_t = lax.broadcasted_iota(jnp.int32, (NSEG, 1), 0) // NUM_LABEL
    G = (lax.broadcasted_iota(jnp.int32, (NUM_TIME, NSEG), 1) // NUM_LABEL
         == lax.broadcasted_iota(jnp.int32, (NUM_TIME, NSEG), 0)).astype(f32)
    GT = (lax.broadcasted_iota(jnp.int32, (NSEG, NUM_TIME), 0) // NUM_LABEL
          == lax.broadcasted_iota(jnp.int32, (NSEG, NUM_TIME), 1)).astype(f32)

    time_cnt = jnp.dot(G, cnt, preferred_element_type=f32)
    tsums = jnp.dot(G, sums, preferred_element_type=f32)
    mean = sums / jnp.maximum(1.0, cnt)
    ttm = tsums / jnp.maximum(1.0, time_cnt)
    ttm_seg = jnp.dot(GT, ttm, preferred_element_type=f32)

    diff = mean - ttm_seg
    msq_seg = cnt * jnp.sum(diff * diff, axis=1, keepdims=True)
    rsq_seg = (sumsq - 2.0 * jnp.sum(mean * sums, axis=1, keepdims=True)
               + cnt * jnp.sum(mean * mean, axis=1, keepdims=True))
    msq_t = jnp.dot(G, msq_seg, preferred_element_type=f32)
    rsq_t = jnp.dot(G, rsq_seg, preferred_element_type=f32)

    testm = (seg_t >= SPLIT).astype(f32)
    test_cnt = jnp.sum(testm * cnt)
    test_sum = jnp.sum(sums * testm, axis=0, keepdims=True)
    test_sumsq = jnp.sum(sumsq * testm)
    test_mean = test_sum / jnp.maximum(1.0, test_cnt)
    test_var = (test_sumsq - 2.0 * jnp.sum(test_mean * test_sum)
                + test_cnt * jnp.sum(test_mean * test_mean)
                ) / jnp.maximum(1.0, test_cnt - 1.0)

    t_iota = lax.broadcasted_iota(jnp.int32, (NUM_TIME, 1), 0)
    tmask = t_iota < SPLIT
    denom = jnp.maximum(1.0, time_cnt - 1.0)
    msq = jnp.where(tmask, msq_t / denom, msq_t)
    rsq = jnp.where(tmask, rsq_t / denom, rsq_t)
    alpha_sq = (test_var - msq) / jnp.maximum(1e-06, rsq)
    alpha = jnp.where(alpha_sq > 0,
                      jnp.sqrt(jnp.where(alpha_sq > 0, alpha_sq, 1.0)), 0.0)

    alpha_seg = jnp.dot(GT, alpha, preferred_element_type=f32)
    train_seg = seg_t < SPLIT
    A = jnp.where(train_seg, alpha_seg, 1.0)
    B = jnp.where(train_seg, (1.0 - alpha_seg) * mean, 0.0)
    tbl_ref[...] = jnp.concatenate(
        [B, A, jnp.zeros((NSEG, STRIDE - D - 1), f32)], axis=1)


_stats = pl.pallas_call(
    _stats_body,
    out_shape=jax.ShapeDtypeStruct((NSEG, STRIDE), jnp.float32),
)


@functools.partial(
    pl.kernel,
    out_type=jax.ShapeDtypeStruct((N * D,), jnp.float32),
    mesh=_mesh,
    scratch_types=[
        pltpu.VMEM((CW,), jnp.float32),
        pltpu.VMEM((C,), jnp.int32),
        pltpu.VMEM((C,), jnp.int32),
        pltpu.VMEM((ACC,), jnp.float32),
        pltpu.VMEM((CW,), jnp.float32),
        pltpu.VMEM((L,), jnp.int32),
        pltpu.VMEM((L,), jnp.float32),
    ],
    compiler_params=_sc_params,
)
def _pass2(x_hbm, labels_hbm, times_hbm, tbl_hbm, out_hbm,
           xb, lb, tb, tblv, ob, sbuf, abuf):
    wid = lax.axis_index("s") * NC + lax.axis_index("c")
    lane = _lane()
    pltpu.sync_copy(tbl_hbm, tblv)

    def chunk_body(i, _):
        j = wid + i * NW

        @pl.when(j < NCHUNK)
        def _():
            pltpu.sync_copy(x_hbm.at[pl.ds(j * CW, CW)], xb)
            pltpu.sync_copy(labels_hbm.at[pl.ds(j * C, C)], lb)
            pltpu.sync_copy(times_hbm.at[pl.ds(j * C, C)], tb)

            def group_body(g, _):
                lv = lb[pl.ds(g * L, L)]
                tv = tb[pl.ds(g * L, L)]
                seg = tv * NUM_LABEL + lv
                sbuf[...] = seg
                abuf[...] = plsc.load_gather(tblv, [seg * STRIDE + D])
                for r in range(L):
                    a = _splat(abuf, r)
                    fb = _splat(sbuf, r) * STRIDE
                    base = g * (L * D) + r * D
                    for k in range(D // L):
                        xv = xb[pl.ds(base + k * L, L)]
                        bv = plsc.load_gather(tblv, [fb + (k * L) + lane])
                        ob[pl.ds(base + k * L, L)] = a * xv + bv
                return 0
            lax.fori_loop(0, C // L, group_body, 0)
            pltpu.sync_copy(ob, out_hbm.at[pl.ds(j * CW, CW)])
        return 0
    lax.fori_loop(0, MAXCH, chunk_body, 0)


def kernel(x, labels, times):
    x_flat = x.reshape(-1)
    partials = _pass1(x_flat, labels, times)
    tbl = _stats(partials.reshape(NW, NSEG, STRIDE))
    out_flat = _pass2(x_flat, labels, times, tbl.reshape(-1))
    return out_flat.reshape(N, D)


# speedup vs baseline: 9.3404x; 1.7396x over previous
import functools

import jax
import jax.numpy as jnp
from jax import lax
from jax.experimental import pallas as pl
from jax.experimental.pallas import tpu as pltpu
from jax.experimental.pallas import tpu_sc as plsc

N = 100000
D = 128
NUM_TIME = 20
NUM_LABEL = 16
SPLIT = 15
NSEG = NUM_TIME * NUM_LABEL
STRIDE = D + 32
ACC = NSEG * STRIDE

NC, NS, L = 2, 16, 16
NW = NC * NS
C = 160
NCHUNK = N // C
CW = C * D
MAXCH = (NCHUNK + NW - 1) // NW

_mesh = plsc.VectorSubcoreMesh(
    core_axis_name="c", subcore_axis_name="s", num_cores=NC, num_subcores=NS)
_sc_params = pltpu.CompilerParams(needs_layout_passes=False)


def _lane():
    return lax.iota(jnp.int32, L)


def _splat(buf, r):
    return plsc.load_gather(buf, [jnp.full((L,), r, jnp.int32)])


@functools.partial(
    pl.kernel,
    out_type=jax.ShapeDtypeStruct((NW, ACC), jnp.float32),
    mesh=_mesh,
    scratch_types=[
        pltpu.VMEM((CW,), jnp.float32),
        pltpu.VMEM((C,), jnp.int32),
        pltpu.VMEM((C,), jnp.int32),
        pltpu.VMEM((ACC,), jnp.float32),
        pltpu.VMEM((C,), jnp.int32),
    ],
    compiler_params=_sc_params,
)
def _pass1(x_hbm, labels_hbm, times_hbm, out_hbm, xb, lb, tb, acc, segb):
    wid = lax.axis_index("s") * NC + lax.axis_index("c")
    lane = _lane()
    ones = jnp.ones((L,), jnp.float32)

    def zero_body(i, _):
        acc[pl.ds(i * L, L)] = jnp.zeros((L,), jnp.float32)
        return 0
    lax.fori_loop(0, ACC // L, zero_body, 0)

    def chunk_body(i, _):
        j = wid + i * NW

        @pl.when(j < NCHUNK)
        def _():
            pltpu.sync_copy(x_hbm.at[pl.ds(j * CW, CW)], xb)
            pltpu.sync_copy(labels_hbm.at[pl.ds(j * C, C)], lb)
            pltpu.sync_copy(times_hbm.at[pl.ds(j * C, C)], tb)

            def seg_body(g, _):
                lv = lb[pl.ds(g * L, L)]
                tv = tb[pl.ds(g * L, L)]
                segb[pl.ds(g * L, L)] = tv * NUM_LABEL + lv
                return 0
            lax.fori_loop(0, C // L, seg_body, 0)

            @plsc.parallel_loop(0, C, unroll=2)
            def row_body(r):
                rv = jnp.broadcast_to(r, (L,))
                fb = plsc.load_gather(segb, [rv]) * STRIDE
                base = r * D
                sq = jnp.zeros((L,), jnp.float32)
                for k in range(D // L):
                    xv = xb[pl.ds(base + k * L, L)]
                    plsc.addupdate_scatter(acc, [fb + (k * L) + lane], xv)
                    sq = sq + xv * xv
                plsc.addupdate_scatter(acc, [fb + D + lane], sq)
                plsc.addupdate_scatter(acc, [fb + (D + L) + lane], ones)
        return 0
    lax.fori_loop(0, MAXCH, chunk_body, 0)

    pltpu.sync_copy(acc, out_hbm.at[wid])


def _stats_body(p_ref, tbl_ref):
    f32 = jnp.float32
    ps = jnp.sum(p_ref[...], axis=0)
    sums = ps[:, :D]
    cnt = ps[:, D + L:D + L + 1]
    sumsq = jnp.sum(ps[:, D:D + L], axis=1, keepdims=True)

    seg_t = lax.broadcasted_iota(jnp.int32, (NSEG, 1), 0) // NUM_LABEL
    G = (lax.broadcasted_iota(jnp.int32, (NUM_TIME, NSEG), 1) // NUM_LABEL
         == lax.broadcasted_iota(jnp.int32, (NUM_TIME, NSEG), 0)).astype(f32)
    GT = (lax.broadcasted_iota(jnp.int32, (NSEG, NUM_TIME), 0) // NUM_LABEL
          == lax.broadcasted_iota(jnp.int32, (NSEG, NUM_TIME), 1)).astype(f32)

    time_cnt = jnp.dot(G, cnt, preferred_element_type=f32)
    tsums = jnp.dot(G, sums, preferred_element_type=f32)
    mean = sums / jnp.maximum(1.0, cnt)
    ttm = tsums / jnp.maximum(1.0, time_cnt)
    ttm_seg = jnp.dot(GT, ttm, preferred_element_type=f32)

    diff = mean - ttm_seg
    msq_seg = cnt * jnp.sum(diff * diff, axis=1, keepdims=True)
    rsq_seg = (sumsq - 2.0 * jnp.sum(mean * sums, axis=1, keepdims=True)
               + cnt * jnp.sum(mean * mean, axis=1, keepdims=True))
    msq_t = jnp.dot(G, msq_seg, preferred_element_type=f32)
    rsq_t = jnp.dot(G, rsq_seg, preferred_element_type=f32)

    testm = (seg_t >= SPLIT).astype(f32)
    test_cnt = jnp.sum(testm * cnt)
    test_sum = jnp.sum(sums * testm, axis=0, keepdims=True)
    test_sumsq = jnp.sum(sumsq * testm)
    test_mean = test_sum / jnp.maximum(1.0, test_cnt)
    test_var = (test_sumsq - 2.0 * jnp.sum(test_mean * test_sum)
                + test_cnt * jnp.sum(test_mean * test_mean)
                ) / jnp.maximum(1.0, test_cnt - 1.0)

    t_iota = lax.broadcasted_iota(jnp.int32, (NUM_TIME, 1), 0)
    tmask = t_iota < SPLIT
    denom = jnp.maximum(1.0, time_cnt - 1.0)
    msq = jnp.where(tmask, msq_t / denom, msq_t)
    rsq = jnp.where(tmask, rsq_t / denom, rsq_t)
    alpha_sq = (test_var - msq) / jnp.maximum(1e-06, rsq)
    alpha = jnp.where(alpha_sq > 0,
                      jnp.sqrt(jnp.where(alpha_sq > 0, alpha_sq, 1.0)), 0.0)

    alpha_seg = jnp.dot(GT, alpha, preferred_element_type=f32)
    train_seg = seg_t < SPLIT
    A = jnp.where(train_seg, alpha_seg, 1.0)
    B = jnp.where(train_seg, (1.0 - alpha_seg) * mean, 0.0)
    tbl_ref[...] = jnp.concatenate(
        [B, A, jnp.zeros((NSEG, STRIDE - D - 1), f32)], axis=1)


_stats = pl.pallas_call(
    _stats_body,
    out_shape=jax.ShapeDtypeStruct((NSEG, STRIDE), jnp.float32),
)


@functools.partial(
    pl.kernel,
    out_type=jax.ShapeDtypeStruct((N * D,), jnp.float32),
    mesh=_mesh,
    scratch_types=[
        pltpu.VMEM((CW,), jnp.float32),
        pltpu.VMEM((C,), jnp.int32),
        pltpu.VMEM((C,), jnp.int32),
        pltpu.VMEM((ACC,), jnp.float32),
        pltpu.VMEM((CW,), jnp.float32),
        pltpu.VMEM((C,), jnp.int32),
        pltpu.VMEM((C,), jnp.float32),
    ],
    compiler_params=_sc_params,
)
def _pass2(x_hbm, labels_hbm, times_hbm, tbl_hbm, out_hbm,
           xb, lb, tb, tblv, ob, segb, ab):
    wid = lax.axis_index("s") * NC + lax.axis_index("c")
    lane = _lane()
    pltpu.sync_copy(tbl_hbm, tblv)

    def chunk_body(i, _):
        j = wid + i * NW

        @pl.when(j < NCHUNK)
        def _():
            pltpu.sync_copy(x_hbm.at[pl.ds(j * CW, CW)], xb)
            pltpu.sync_copy(labels_hbm.at[pl.ds(j * C, C)], lb)
            pltpu.sync_copy(times_hbm.at[pl.ds(j * C, C)], tb)

            def seg_body(g, _):
                lv = lb[pl.ds(g * L, L)]
                tv = tb[pl.ds(g * L, L)]
                seg = tv * NUM_LABEL + lv
                segb[pl.ds(g * L, L)] = seg
                ab[pl.ds(g * L, L)] = plsc.load_gather(tblv, [seg * STRIDE + D])
                return 0
            lax.fori_loop(0, C // L, seg_body, 0)

            @plsc.parallel_loop(0, C, unroll=2)
            def row_body(r):
                rv = jnp.broadcast_to(r, (L,))
                a = plsc.load_gather(ab, [rv])
                fb = plsc.load_gather(segb, [rv]) * STRIDE
                base = r * D
                for k in range(D // L):
                    xv = xb[pl.ds(base + k * L, L)]
                    bv = plsc.load_gather(tblv, [fb + (k * L) + lane])
                    ob[pl.ds(base + k * L, L)] = a * xv + bv
            pltpu.sync_copy(ob, out_hbm.at[pl.ds(j * CW, CW)])
        return 0
    lax.fori_loop(0, MAXCH, chunk_body, 0)


def kernel(x, labels, times):
    x_flat = x.reshape(-1)
    partials = _pass1(x_flat, labels, times)
    tbl = _stats(partials.reshape(NW, NSEG, STRIDE))
    out_flat = _pass2(x_flat, labels, times, tbl.reshape(-1))
    return out_flat.reshape(N, D)


# speedup vs baseline: 9.3498x; 1.0010x over previous
import functools

import jax
import jax.numpy as jnp
from jax import lax
from jax.experimental import pallas as pl
from jax.experimental.pallas import tpu as pltpu
from jax.experimental.pallas import tpu_sc as plsc

N = 100000
D = 128
NUM_TIME = 20
NUM_LABEL = 16
SPLIT = 15
NSEG = NUM_TIME * NUM_LABEL
STRIDE = D + 32
ACC = NSEG * STRIDE

NC, NS, L = 2, 16, 16
NW = NC * NS
C = 160
NCHUNK = N // C
CW = C * D
MAXCH = (NCHUNK + NW - 1) // NW

_mesh = plsc.VectorSubcoreMesh(
    core_axis_name="c", subcore_axis_name="s", num_cores=NC, num_subcores=NS)
_sc_params = pltpu.CompilerParams(needs_layout_passes=False)


def _lane():
    return lax.iota(jnp.int32, L)


def _splat(buf, r):
    return plsc.load_gather(buf, [jnp.full((L,), r, jnp.int32)])


@functools.partial(
    pl.kernel,
    out_type=jax.ShapeDtypeStruct((NW, ACC), jnp.float32),
    mesh=_mesh,
    scratch_types=[
        pltpu.VMEM((CW,), jnp.float32),
        pltpu.VMEM((C,), jnp.int32),
        pltpu.VMEM((C,), jnp.int32),
        pltpu.VMEM((ACC,), jnp.float32),
        pltpu.VMEM((C,), jnp.int32),
    ],
    compiler_params=_sc_params,
)
def _pass1(x_hbm, labels_hbm, times_hbm, out_hbm, xb, lb, tb, acc, segb):
    wid = lax.axis_index("s") * NC + lax.axis_index("c")
    lane = _lane()
    ones = jnp.ones((L,), jnp.float32)

    def zero_body(i, _):
        acc[pl.ds(i * L, L)] = jnp.zeros((L,), jnp.float32)
        return 0
    lax.fori_loop(0, ACC // L, zero_body, 0)

    def chunk_body(i, _):
        j = wid + i * NW

        @pl.when(j < NCHUNK)
        def _():
            pltpu.sync_copy(x_hbm.at[pl.ds(j * CW, CW)], xb)
            pltpu.sync_copy(labels_hbm.at[pl.ds(j * C, C)], lb)
            pltpu.sync_copy(times_hbm.at[pl.ds(j * C, C)], tb)

            def seg_body(g, _):
                lv = lb[pl.ds(g * L, L)]
                tv = tb[pl.ds(g * L, L)]
                segb[pl.ds(g * L, L)] = tv * NUM_LABEL + lv
                return 0
            lax.fori_loop(0, C // L, seg_body, 0)

            @plsc.parallel_loop(0, C, unroll=4)
            def row_body(r):
                rv = jnp.broadcast_to(r, (L,))
                fb = plsc.load_gather(segb, [rv]) * STRIDE
                base = r * D
                sq = jnp.zeros((L,), jnp.float32)
                for k in range(D // L):
                    xv = xb[pl.ds(base + k * L, L)]
                    plsc.addupdate_scatter(acc, [fb + (k * L) + lane], xv)
                    sq = sq + xv * xv
                plsc.addupdate_scatter(acc, [fb + D + lane], sq)
                plsc.addupdate_scatter(acc, [fb + (D + L) + lane], ones)
        return 0
    lax.fori_loop(0, MAXCH, chunk_body, 0)

    pltpu.sync_copy(acc, out_hbm.at[wid])


def _stats_body(p_ref, tbl_ref):
    f32 = jnp.float32
    ps = jnp.sum(p_ref[...], axis=0)
    sums = ps[:, :D]
    cnt = ps[:, D + L:D + L + 1]
    sumsq = jnp.sum(ps[:, D:D + L], axis=1, keepdims=True)

    seg_t = lax.broadcasted_iota(jnp.int32, (NSEG, 1), 0) // NUM_LABEL
    G = (lax.broadcasted_iota(jnp.int32, (NUM_TIME, NSEG), 1) // NUM_LABEL
         == lax.broadcasted_iota(jnp.int32, (NUM_TIME, NSEG), 0)).astype(f32)
    GT = (lax.broadcasted_iota(jnp.int32, (NSEG, NUM_TIME), 0) // NUM_LABEL
          == lax.broadcasted_iota(jnp.int32, (NSEG, NUM_TIME), 1)).astype(f32)

    time_cnt = jnp.dot(G, cnt, preferred_element_type=f32)
    tsums = jnp.dot(G, sums, preferred_element_type=f32)
    mean = sums / jnp.maximum(1.0, cnt)
    ttm = tsums / jnp.maximum(1.0, time_cnt)
    ttm_seg = jnp.dot(GT, ttm, preferred_element_type=f32)

    diff = mean - ttm_seg
    msq_seg = cnt * jnp.sum(diff * diff, axis=1, keepdims=True)
    rsq_seg = (sumsq - 2.0 * jnp.sum(mean * sums, axis=1, keepdims=True)
               + cnt * jnp.sum(mean * mean, axis=1, keepdims=True))
    msq_t = jnp.dot(G, msq_seg, preferred_element_type=f32)
    rsq_t = jnp.dot(G, rsq_seg, preferred_element_type=f32)

    testm = (seg_t >= SPLIT).astype(f32)
    test_cnt = jnp.sum(testm * cnt)
    test_sum = jnp.sum(sums * testm, axis=0, keepdims=True)
    test_sumsq = jnp.sum(sumsq * testm)
    test_mean = test_sum / jnp.maximum(1.0, test_cnt)
    test_var = (test_sumsq - 2.0 * jnp.sum(test_mean * test_sum)
                + test_cnt * jnp.sum(test_mean * test_mean)
                ) / jnp.maximum(1.0, test_cnt - 1.0)

    t_iota = lax.broadcasted_iota(jnp.int32, (NUM_TIME, 1), 0)
    tmask = t_iota < SPLIT
    denom = jnp.maximum(1.0, time_cnt - 1.0)
    msq = jnp.where(tmask, msq_t / denom, msq_t)
    rsq = jnp.where(tmask, rsq_t / denom, rsq_t)
    alpha_sq = (test_var - msq) / jnp.maximum(1e-06, rsq)
    alpha = jnp.where(alpha_sq > 0,
                      jnp.sqrt(jnp.where(alpha_sq > 0, alpha_sq, 1.0)), 0.0)

    alpha_seg = jnp.dot(GT, alpha, preferred_element_type=f32)
    train_seg = seg_t < SPLIT
    A = jnp.where(train_seg, alpha_seg, 1.0)
    B = jnp.where(train_seg, (1.0 - alpha_seg) * mean, 0.0)
    tbl_ref[...] = jnp.concatenate(
        [B, A, jnp.zeros((NSEG, STRIDE - D - 1), f32)], axis=1)


_stats = pl.pallas_call(
    _stats_body,
    out_shape=jax.ShapeDtypeStruct((NSEG, STRIDE), jnp.float32),
)


@functools.partial(
    pl.kernel,
    out_type=jax.ShapeDtypeStruct((N * D,), jnp.float32),
    mesh=_mesh,
    scratch_types=[
        pltpu.VMEM((CW,), jnp.float32),
        pltpu.VMEM((C,), jnp.int32),
        pltpu.VMEM((C,), jnp.int32),
        pltpu.VMEM((ACC,), jnp.float32),
        pltpu.VMEM((CW,), jnp.float32),
        pltpu.VMEM((C,), jnp.int32),
        pltpu.VMEM((C,), jnp.float32),
    ],
    compiler_params=_sc_params,
)
def _pass2(x_hbm, labels_hbm, times_hbm, tbl_hbm, out_hbm,
           xb, lb, tb, tblv, ob, segb, ab):
    wid = lax.axis_index("s") * NC + lax.axis_index("c")
    lane = _lane()
    pltpu.sync_copy(tbl_hbm, tblv)

    def chunk_body(i, _):
        j = wid + i * NW

        @pl.when(j < NCHUNK)
        def _():
            pltpu.sync_copy(x_hbm.at[pl.ds(j * CW, CW)], xb)
            pltpu.sync_copy(labels_hbm.at[pl.ds(j * C, C)], lb)
            pltpu.sync_copy(times_hbm.at[pl.ds(j * C, C)], tb)

            def seg_body(g, _):
                lv = lb[pl.ds(g * L, L)]
                tv = tb[pl.ds(g * L, L)]
                seg = tv * NUM_LABEL + lv
                segb[pl.ds(g * L, L)] = seg
                ab[pl.ds(g * L, L)] = plsc.load_gather(tblv, [seg * STRIDE + D])
                return 0
            lax.fori_loop(0, C // L, seg_body, 0)

            @plsc.parallel_loop(0, C, unroll=4)
            def row_body(r):
                rv = jnp.broadcast_to(r, (L,))
                a = plsc.load_gather(ab, [rv])
                fb = plsc.load_gather(segb, [rv]) * STRIDE
                base = r * D
                for k in range(D // L):
                    xv = xb[pl.ds(base + k * L, L)]
                    bv = plsc.load_gather(tblv, [fb + (k * L) + lane])
                    ob[pl.ds(base + k * L, L)] = a * xv + bv
            pltpu.sync_copy(ob, out_hbm.at[pl.ds(j * CW, CW)])
        return 0
    lax.fori_loop(0, MAXCH, chunk_body, 0)


def kernel(x, labels, times):
    x_flat = x.reshape(-1)
    partials = _pass1(x_flat, labels, times)
    tbl = _stats(partials.reshape(NW, NSEG, STRIDE))
    out_flat = _pass2(x_flat, labels, times, tbl.reshape(-1))
    return out_flat.reshape(N, D)


# speedup vs baseline: 16.5114x; 1.7659x over previous
import functools

import jax
import jax.numpy as jnp
from jax import lax
from jax.experimental import pallas as pl
from jax.experimental.pallas import tpu as pltpu
from jax.experimental.pallas import tpu_sc as plsc

N = 100000
D = 128
NUM_TIME = 20
NUM_LABEL = 16
SPLIT = 15
NSEG = NUM_TIME * NUM_LABEL
STRIDE = D + 32
ACC = NSEG * STRIDE
TBLW = D + 2
TBLN = NSEG * TBLW

NC, NS, L = 2, 16, 16
NW = NC * NS
C = 160
NCHUNK = N // C
CW = C * D
MAXCH = (NCHUNK + NW - 1) // NW

_mesh = plsc.VectorSubcoreMesh(
    core_axis_name="c", subcore_axis_name="s", num_cores=NC, num_subcores=NS)
_sc_params = pltpu.CompilerParams(needs_layout_passes=False)


def _lane():
    return lax.iota(jnp.int32, L)


@functools.partial(
    pl.kernel,
    out_type=jax.ShapeDtypeStruct((NW, ACC), jnp.float32),
    mesh=_mesh,
    scratch_types=[
        pltpu.VMEM((CW,), jnp.float32),
        pltpu.VMEM((CW,), jnp.float32),
        pltpu.VMEM((C,), jnp.int32),
        pltpu.VMEM((C,), jnp.int32),
        pltpu.VMEM((C,), jnp.int32),
        pltpu.VMEM((C,), jnp.int32),
        pltpu.VMEM((ACC,), jnp.float32),
        pltpu.VMEM((C,), jnp.int32),
        pltpu.SemaphoreType.DMA,
        pltpu.SemaphoreType.DMA,
    ],
    compiler_params=_sc_params,
)
def _pass1(x_hbm, labels_hbm, times_hbm, out_hbm,
           xb0, xb1, lb0, lb1, tb0, tb1, acc, segb, si0, si1):
    wid = lax.axis_index("s") * NC + lax.axis_index("c")
    lane = _lane()
    ones = jnp.ones((L,), jnp.float32)

    def fire_in(j, xb, lb, tb, sem):
        pltpu.async_copy(x_hbm.at[pl.ds(j * CW, CW)], xb, sem)
        pltpu.async_copy(labels_hbm.at[pl.ds(j * C, C)], lb, sem)
        pltpu.async_copy(times_hbm.at[pl.ds(j * C, C)], tb, sem)

    def wait_in(j, xb, lb, tb, sem):
        pltpu.make_async_copy(x_hbm.at[pl.ds(j * CW, CW)], xb, sem).wait()
        pltpu.make_async_copy(labels_hbm.at[pl.ds(j * C, C)], lb, sem).wait()
        pltpu.make_async_copy(times_hbm.at[pl.ds(j * C, C)], tb, sem).wait()

    def compute(xb, lb, tb):
        def seg_body(g, _):
            lv = lb[pl.ds(g * L, L)]
            tv = tb[pl.ds(g * L, L)]
            segb[pl.ds(g * L, L)] = tv * NUM_LABEL + lv
            return 0
        lax.fori_loop(0, C // L, seg_body, 0)

        @plsc.parallel_loop(0, C, unroll=2)
        def row_body(r):
            rv = jnp.broadcast_to(r, (L,))
            fb = plsc.load_gather(segb, [rv]) * STRIDE
            base = r * D
            sq = jnp.zeros((L,), jnp.float32)
            for k in range(D // L):
                xv = xb[pl.ds(base + k * L, L)]
                plsc.addupdate_scatter(acc, [fb + (k * L) + lane], xv)
                sq = sq + xv * xv
            plsc.addupdate_scatter(acc, [fb + D + lane], sq)
            plsc.addupdate_scatter(acc, [fb + (D + L) + lane], ones)

    fire_in(wid, xb0, lb0, tb0, si0)

    def zero_body(i, _):
        acc[pl.ds(i * L, L)] = jnp.zeros((L,), jnp.float32)
        return 0
    lax.fori_loop(0, ACC // L, zero_body, 0)

    def chunk_pair(i, _):
        j0 = wid + (2 * i) * NW
        j1 = j0 + NW
        jn = j0 + 2 * NW

        @pl.when(j1 < NCHUNK)
        def _():
            fire_in(j1, xb1, lb1, tb1, si1)

        @pl.when(j0 < NCHUNK)
        def _():
            wait_in(j0, xb0, lb0, tb0, si0)
            compute(xb0, lb0, tb0)

        @pl.when(jn < NCHUNK)
        def _():
            fire_in(jn, xb0, lb0, tb0, si0)

        @pl.when(j1 < NCHUNK)
        def _():
            wait_in(j1, xb1, lb1, tb1, si1)
            compute(xb1, lb1, tb1)
        return 0
    lax.fori_loop(0, MAXCH // 2, chunk_pair, 0)

    pltpu.sync_copy(acc, out_hbm.at[wid])


def _stats_body(p_ref, tbl_ref):
    f32 = jnp.float32
    ps = jnp.sum(p_ref[...], axis=0)
    sums = ps[:, :D]
    cnt = ps[:, D + L:D + L + 1]
    sumsq = jnp.sum(ps[:, D:D + L], axis=1, keepdims=True)

    seg_t = lax.broadcasted_iota(jnp.int32, (NSEG, 1), 0) // NUM_LABEL
    G = (lax.broadcasted_iota(jnp.int32, (NUM_TIME, NSEG), 1) // NUM_LABEL
         == lax.broadcasted_iota(jnp.int32, (NUM_TIME, NSEG), 0)).astype(f32)
    GT = (lax.broadcasted_iota(jnp.int32, (NSEG, NUM_TIME), 0) // NUM_LABEL
          == lax.broadcasted_iota(jnp.int32, (NSEG, NUM_TIME), 1)).astype(f32)

    time_cnt = jnp.dot(G, cnt, preferred_element_type=f32)
    tsums = jnp.dot(G, sums, preferred_element_type=f32)
    mean = sums / jnp.maximum(1.0, cnt)
    ttm = tsums / jnp.maximum(1.0, time_cnt)
    ttm_seg = jnp.dot(GT, ttm, preferred_element_type=f32)

    diff = mean - ttm_seg
    msq_seg = cnt * jnp.sum(diff * diff, axis=1, keepdims=True)
    rsq_seg = (sumsq - 2.0 * jnp.sum(mean * sums, axis=1, keepdims=True)
               + cnt * jnp.sum(mean * mean, axis=1, keepdims=True))
    msq_t = jnp.dot(G, msq_seg, preferred_element_type=f32)
    rsq_t = jnp.dot(G, rsq_seg, preferred_element_type=f32)

    testm = (seg_t >= SPLIT).astype(f32)
    test_cnt = jnp.sum(testm * cnt)
    test_sum = jnp.sum(sums * testm, axis=0, keepdims=True)
    test_sumsq = jnp.sum(sumsq * testm)
    test_mean = test_sum / jnp.maximum(1.0, test_cnt)
    test_var = (test_sumsq - 2.0 * jnp.sum(test_mean * test_sum)
                + test_cnt * jnp.sum(test_mean * test_mean)
                ) / jnp.maximum(1.0, test_cnt - 1.0)

    t_iota = lax.broadcasted_iota(jnp.int32, (NUM_TIME, 1), 0)
    tmask = t_iota < SPLIT
    denom = jnp.maximum(1.0, time_cnt - 1.0)
    msq = jnp.where(tmask, msq_t / denom, msq_t)
    rsq = jnp.where(tmask, rsq_t / denom, rsq_t)
    alpha_sq = (test_var - msq) / jnp.maximum(1e-06, rsq)
    alpha = jnp.where(alpha_sq > 0,
                      jnp.sqrt(jnp.where(alpha_sq > 0, alpha_sq, 1.0)), 0.0)

    alpha_seg = jnp.dot(GT, alpha, preferred_element_type=f32)
    train_seg = seg_t < SPLIT
    A = jnp.where(train_seg, alpha_seg, 1.0)
    B = jnp.where(train_seg, (1.0 - alpha_seg) * mean, 0.0)
    tbl_ref[...] = jnp.concatenate(
        [B, A, jnp.zeros((NSEG, TBLW - D - 1), f32)], axis=1)


_stats = pl.pallas_call(
    _stats_body,
    out_shape=jax.ShapeDtypeStruct((NSEG, TBLW), jnp.float32),
)


@functools.partial(
    pl.kernel,
    out_type=jax.ShapeDtypeStruct((N * D,), jnp.float32),
    mesh=_mesh,
    scratch_types=[
        pltpu.VMEM((CW,), jnp.float32),
        pltpu.VMEM((CW,), jnp.float32),
        pltpu.VMEM((C,), jnp.int32),
        pltpu.VMEM((C,), jnp.int32),
        pltpu.VMEM((C,), jnp.int32),
        pltpu.VMEM((C,), jnp.int32),
        pltpu.VMEM((TBLN,), jnp.float32),
        pltpu.VMEM((CW,), jnp.float32),
        pltpu.VMEM((CW,), jnp.float32),
        pltpu.VMEM((C,), jnp.int32),
        pltpu.VMEM((C,), jnp.float32),
        pltpu.SemaphoreType.DMA,
        pltpu.SemaphoreType.DMA,
        pltpu.SemaphoreType.DMA,
        pltpu.SemaphoreType.DMA,
    ],
    compiler_params=_sc_params,
)
def _pass2(x_hbm, labels_hbm, times_hbm, tbl_hbm, out_hbm,
           xb0, xb1, lb0, lb1, tb0, tb1, tblv, ob0, ob1, segb, ab,
           si0, si1, so0, so1):
    wid = lax.axis_index("s") * NC + lax.axis_index("c")
    lane = _lane()

    def fire_in(j, xb, lb, tb, sem):
        pltpu.async_copy(x_hbm.at[pl.ds(j * CW, CW)], xb, sem)
        pltpu.async_copy(labels_hbm.at[pl.ds(j * C, C)], lb, sem)
        pltpu.async_copy(times_hbm.at[pl.ds(j * C, C)], tb, sem)

    def wait_in(j, xb, lb, tb, sem):
        pltpu.make_async_copy(x_hbm.at[pl.ds(j * CW, CW)], xb, sem).wait()
        pltpu.make_async_copy(labels_hbm.at[pl.ds(j * C, C)], lb, sem).wait()
        pltpu.make_async_copy(times_hbm.at[pl.ds(j * C, C)], tb, sem).wait()

    def wait_out(j, ob, sem):
        pltpu.make_async_copy(ob, out_hbm.at[pl.ds(j * CW, CW)], sem).wait()

    def compute(j, xb, lb, tb, ob, osem):
        def seg_body(g, _):
            lv = lb[pl.ds(g * L, L)]
            tv = tb[pl.ds(g * L, L)]
            seg = tv * NUM_LABEL + lv
            segb[pl.ds(g * L, L)] = seg
            ab[pl.ds(g * L, L)] = plsc.load_gather(tblv, [seg * TBLW + D])
            return 0
        lax.fori_loop(0, C // L, seg_body, 0)

        @plsc.parallel_loop(0, C, unroll=2)
        def row_body(r):
            rv = jnp.broadcast_to(r, (L,))
            a = plsc.load_gather(ab, [rv])
            fb = plsc.load_gather(segb, [rv]) * TBLW
            base = r * D
            for k in range(D // L):
                xv = xb[pl.ds(base + k * L, L)]
                bv = plsc.load_gather(tblv, [fb + (k * L) + lane])
                ob[pl.ds(base + k * L, L)] = a * xv + bv

        pltpu.async_copy(ob, out_hbm.at[pl.ds(j * CW, CW)], osem)

    fire_in(wid, xb0, lb0, tb0, si0)
    pltpu.sync_copy(tbl_hbm, tblv)

    def chunk_pair(i, _):
        j0 = wid + (2 * i) * NW
        j1 = j0 + NW
        jn = j0 + 2 * NW

        @pl.when(j1 < NCHUNK)
        def _():
            fire_in(j1, xb1, lb1, tb1, si1)

        @pl.when((i >= 1) & (j0 - 2 * NW < NCHUNK))
        def _():
            wait_out(j0 - 2 * NW, ob0, so0)

        @pl.when(j0 < NCHUNK)
        def _():
            wait_in(j0, xb0, lb0, tb0, si0)
            compute(j0, xb0, lb0, tb0, ob0, so0)

        @pl.when(jn < NCHUNK)
        def _():
            fire_in(jn, xb0, lb0, tb0, si0)

        @pl.when((i >= 1) & (j1 - 2 * NW < NCHUNK))
        def _():
            wait_out(j1 - 2 * NW, ob1, so1)

        @pl.when(j1 < NCHUNK)
        def _():
            wait_in(j1, xb1, lb1, tb1, si1)
            compute(j1, xb1, lb1, tb1, ob1, so1)
        return 0
    lax.fori_loop(0, MAXCH // 2, chunk_pair, 0)

    jl0 = wid + (MAXCH - 2) * NW
    jl1 = wid + (MAXCH - 1) * NW

    @pl.when(jl0 < NCHUNK)
    def _():
        wait_out(jl0, ob0, so0)

    @pl.when(jl1 < NCHUNK)
    def _():
        wait_out(jl1, ob1, so1)


def kernel(x, labels, times):
    x_flat = x.reshape(-1)
    partials = _pass1(x_flat, labels, times)
    tbl = _stats(partials.reshape(NW, NSEG, STRIDE))
    out_flat = _pass2(x_flat, labels, times, tbl.reshape(-1))
    return out_flat.reshape(N, D)
